# baseline (device time: 226542 ns/iter reference)
import functools

import jax
import jax.numpy as jnp
from jax import lax
from jax.experimental import pallas as pl
from jax.experimental.pallas import tpu as pltpu

N_DEV = 32


def kernel(x, router_W, route_idx, expert_W):
    T, D = x.shape
    E_local, _, H = expert_W.shape
    CH = T // N_DEV
    n_hops = N_DEV - 1

    def body(x_ref, rw_ref, idx_ref, ew_ref, out_ref,
             comm_ref, rs_send_sems, rs_recv_sems, ag_send_sems, ag_recv_sems):
        my = lax.axis_index("i")
        left = lax.rem(my - 1 + N_DEV, N_DEV)
        right = lax.rem(my + 1, N_DEV)

        def neighbor_barrier(sem):
            for nbr in (left, right):
                pl.semaphore_signal(
                    sem, inc=1, device_id=(nbr,),
                    device_id_type=pl.DeviceIdType.MESH,
                )
            pl.semaphore_wait(sem, 2)

        barrier_sem = pltpu.get_barrier_semaphore()
        neighbor_barrier(barrier_sem)

        xv = x_ref[:, :]
        scores = jnp.dot(xv, rw_ref[:, :],
                         preferred_element_type=jnp.float32)
        m = jnp.max(scores, axis=1, keepdims=True)
        p = jnp.exp(scores - m)
        p = p / jnp.sum(p, axis=1, keepdims=True)

        e0 = idx_ref[:, 0:1]
        e1 = idx_ref[:, 1:2]
        eid = lax.broadcasted_iota(jnp.int32, scores.shape, 1)
        g0 = jnp.sum(jnp.where(eid == e0, p, 0.0), axis=1, keepdims=True)
        g1 = jnp.sum(jnp.where(eid == e1, p, 0.0), axis=1, keepdims=True)
        gs = g0 + g1
        w0 = g0 / gs
        w1 = g1 / gs

        acc = jnp.zeros((T, H), jnp.float32)
        for le in range(E_local):
            ge = my * E_local + le
            w = jnp.where(e0 == ge, w0, 0.0) + jnp.where(e1 == ge, w1, 0.0)
            acc = acc + jnp.dot(w * xv, ew_ref[le],
                                preferred_element_type=jnp.float32)
        out_ref[:, :] = acc

        for s in range(n_hops):
            cs = lax.rem(my - s + 2 * N_DEV, N_DEV)
            rdma = pltpu.make_async_remote_copy(
                src_ref=out_ref.at[pl.ds(cs * CH, CH), :],
                dst_ref=comm_ref.at[s],
                send_sem=rs_send_sems.at[s],
                recv_sem=rs_recv_sems.at[s],
                device_id=(right,),
                device_id_type=pl.DeviceIdType.MESH,
            )
            rdma.start()
            rdma.wait()
            cr = lax.rem(my - s - 1 + 2 * N_DEV, N_DEV)
            out_ref[pl.ds(cr * CH, CH), :] = (
                out_ref[pl.ds(cr * CH, CH), :] + comm_ref[s]
            )

        neighbor_barrier(barrier_sem)

        for t in range(n_hops):
            c = lax.rem(my + 1 - t + 2 * N_DEV, N_DEV)
            rdma = pltpu.make_async_remote_copy(
                src_ref=out_ref.at[pl.ds(c * CH, CH), :],
                dst_ref=out_ref.at[pl.ds(c * CH, CH), :],
                send_sem=ag_send_sems.at[t],
                recv_sem=ag_recv_sems.at[t],
                device_id=(right,),
                device_id_type=pl.DeviceIdType.MESH,
            )
            rdma.start()
            rdma.wait()

    return pl.pallas_call(
        body,
        out_shape=jax.ShapeDtypeStruct((T, H), jnp.float32),
        in_specs=[
            pl.BlockSpec(memory_space=pltpu.VMEM),
            pl.BlockSpec(memory_space=pltpu.VMEM),
            pl.BlockSpec(memory_space=pltpu.VMEM),
            pl.BlockSpec(memory_space=pltpu.VMEM),
        ],
        out_specs=pl.BlockSpec(memory_space=pltpu.VMEM),
        scratch_shapes=[
            pltpu.VMEM((N_DEV - 1, CH, H), jnp.float32),
            pltpu.SemaphoreType.DMA((N_DEV - 1,)),
            pltpu.SemaphoreType.DMA((N_DEV - 1,)),
            pltpu.SemaphoreType.DMA((N_DEV - 1,)),
            pltpu.SemaphoreType.DMA((N_DEV - 1,)),
        ],
        compiler_params=pltpu.CompilerParams(collective_id=0),
    )(x, router_W, route_idx, expert_W)


# device time: 136583 ns/iter; 1.6586x vs baseline; 1.6586x over previous
import jax
import jax.numpy as jnp
from jax import lax
from jax.experimental import pallas as pl
from jax.experimental.pallas import tpu as pltpu

N_DEV = 32
BITS = (0, 3, 1, 2, 4)
N_STEPS = len(BITS)
SIZES = tuple(512 >> s for s in range(N_STEPS))
SLOTS = tuple(sum(SIZES[:s]) for s in range(N_STEPS))


def kernel(x, router_W, route_idx, expert_W):
    T, D = x.shape
    E_local, _, H = expert_W.shape

    def body(x_ref, rw_ref, idx_ref, ew_ref, out_ref,
             comm_ref, rs_send_sems, rs_recv_sems, ag_send_sems, ag_recv_sems):
        my = lax.axis_index("i")
        partners = [lax.bitwise_xor(my, 1 << b) for b in BITS]
        bits = [lax.bitwise_and(lax.shift_right_logical(my, b), 1)
                for b in BITS]

        def all_partner_barrier(sem):
            for p in partners:
                pl.semaphore_signal(
                    sem, inc=1, device_id=(p,),
                    device_id_type=pl.DeviceIdType.MESH,
                )
            pl.semaphore_wait(sem, N_STEPS)

        barrier_sem = pltpu.get_barrier_semaphore()
        all_partner_barrier(barrier_sem)

        xv = x_ref[:, :]
        scores = jnp.dot(xv, rw_ref[:, :],
                         preferred_element_type=jnp.float32)
        m = jnp.max(scores, axis=1, keepdims=True)
        p = jnp.exp(scores - m)
        p = p / jnp.sum(p, axis=1, keepdims=True)

        e0 = idx_ref[:, 0:1]
        e1 = idx_ref[:, 1:2]
        eid = lax.broadcasted_iota(jnp.int32, scores.shape, 1)
        g0 = jnp.sum(jnp.where(eid == e0, p, 0.0), axis=1, keepdims=True)
        g1 = jnp.sum(jnp.where(eid == e1, p, 0.0), axis=1, keepdims=True)
        gs = g0 + g1
        w0 = g0 / gs
        w1 = g1 / gs

        acc = jnp.zeros((T, H), jnp.float32)
        for le in range(E_local):
            ge = my * E_local + le
            w = jnp.where(e0 == ge, w0, 0.0) + jnp.where(e1 == ge, w1, 0.0)
            acc = acc + jnp.dot(w * xv, ew_ref[le],
                                preferred_element_type=jnp.float32)
        out_ref[:, :] = acc

        base = jnp.int32(0)
        rs_rdmas = []
        for s in range(N_STEPS):
            h = SIZES[s]
            mb = bits[s]
            keep_off = base + mb * h
            send_off = base + (1 - mb) * h
            rdma = pltpu.make_async_remote_copy(
                src_ref=out_ref.at[pl.ds(send_off, h), :],
                dst_ref=comm_ref.at[pl.ds(SLOTS[s], h), :],
                send_sem=rs_send_sems.at[s],
                recv_sem=rs_recv_sems.at[s],
                device_id=(partners[s],),
                device_id_type=pl.DeviceIdType.MESH,
            )
            rdma.start()
            rdma.wait_recv()
            out_ref[pl.ds(keep_off, h), :] = (
                out_ref[pl.ds(keep_off, h), :]
                + comm_ref[pl.ds(SLOTS[s], h), :]
            )
            rs_rdmas.append(rdma)
            base = keep_off
        for rdma in rs_rdmas:
            rdma.wait_send()

        all_partner_barrier(barrier_sem)

        ag_rdmas = []
        for s in reversed(range(N_STEPS)):
            h = SIZES[s]
            rdma = pltpu.make_async_remote_copy(
                src_ref=out_ref.at[pl.ds(base, h), :],
                dst_ref=out_ref.at[pl.ds(base, h), :],
                send_sem=ag_send_sems.at[s],
                recv_sem=ag_recv_sems.at[s],
                device_id=(partners[s],),
                device_id_type=pl.DeviceIdType.MESH,
            )
            rdma.start()
            rdma.wait_recv()
            ag_rdmas.append(rdma)
            base = base - bits[s] * h
        for rdma in ag_rdmas:
            rdma.wait_send()

    return pl.pallas_call(
        body,
        out_shape=jax.ShapeDtypeStruct((T, H), jnp.float32),
        in_specs=[
            pl.BlockSpec(memory_space=pltpu.VMEM),
            pl.BlockSpec(memory_space=pltpu.VMEM),
            pl.BlockSpec(memory_space=pltpu.VMEM),
            pl.BlockSpec(memory_space=pltpu.VMEM),
        ],
        out_specs=pl.BlockSpec(memory_space=pltpu.VMEM),
        scratch_shapes=[
            pltpu.VMEM((sum(SIZES), H), jnp.float32),
            pltpu.SemaphoreType.DMA((N_STEPS,)),
            pltpu.SemaphoreType.DMA((N_STEPS,)),
            pltpu.SemaphoreType.DMA((N_STEPS,)),
            pltpu.SemaphoreType.DMA((N_STEPS,)),
        ],
        compiler_params=pltpu.CompilerParams(collective_id=0),
    )(x, router_W, route_idx, expert_W)


# device time: 128089 ns/iter; 1.7686x vs baseline; 1.0663x over previous
import jax
import jax.numpy as jnp
from jax import lax
from jax.experimental import pallas as pl
from jax.experimental.pallas import tpu as pltpu

N_DEV = 32
BITS = (0, 3, 1, 2, 4)
N_STEPS = len(BITS)
SIZES = tuple(512 >> s for s in range(N_STEPS))
SLOTS = tuple(sum(SIZES[:s]) for s in range(N_STEPS))


def kernel(x, router_W, route_idx, expert_W):
    T, D = x.shape
    E_local, _, H = expert_W.shape

    def body(x_ref, rw_ref, idx_ref, ew_ref, out_ref,
             comm_ref, rs_send_sems, rs_recv_sems, ag_send_sems, ag_recv_sems):
        my = lax.axis_index("i")
        partners = [lax.bitwise_xor(my, 1 << b) for b in BITS]
        bits = [lax.bitwise_and(lax.shift_right_logical(my, b), 1)
                for b in BITS]

        def all_partner_barrier(sem):
            for p in partners:
                pl.semaphore_signal(
                    sem, inc=1, device_id=(p,),
                    device_id_type=pl.DeviceIdType.MESH,
                )
            pl.semaphore_wait(sem, N_STEPS)

        barrier_sem = pltpu.get_barrier_semaphore()
        all_partner_barrier(barrier_sem)

        import os as _os
        probe = _os.environ.get("PROBE", "")

        xv = x_ref[:, :]
        scores = jnp.dot(xv, rw_ref[:, :],
                         preferred_element_type=jnp.float32)
        m = jnp.max(scores, axis=1, keepdims=True)
        p = jnp.exp(scores - m)
        p = p / jnp.sum(p, axis=1, keepdims=True)

        e0 = idx_ref[:, 0:1]
        e1 = idx_ref[:, 1:2]
        eid = lax.broadcasted_iota(jnp.int32, scores.shape, 1)
        g0 = jnp.sum(jnp.where(eid == e0, p, 0.0), axis=1, keepdims=True)
        g1 = jnp.sum(jnp.where(eid == e1, p, 0.0), axis=1, keepdims=True)
        gs = g0 + g1
        w0 = g0 / gs
        w1 = g1 / gs

        if probe == "comm_only":
            out_ref[:, :] = jnp.broadcast_to(w0, (T, H))
        else:
            acc = jnp.zeros((T, H), jnp.float32)
            for le in range(E_local):
                ge = my * E_local + le
                w = (jnp.where(e0 == ge, w0, 0.0)
                     + jnp.where(e1 == ge, w1, 0.0))
                acc = acc + jnp.dot(w * xv, ew_ref[le],
                                    preferred_element_type=jnp.float32)
            out_ref[:, :] = acc

        if probe == "compute_only":
            return

        base = jnp.int32(0)
        rs_rdmas = []
        for s in range(N_STEPS):
            h = SIZES[s]
            mb = bits[s]
            keep_off = base + mb * h
            send_off = base + (1 - mb) * h
            rdma = pltpu.make_async_remote_copy(
                src_ref=out_ref.at[pl.ds(send_off, h), :],
                dst_ref=comm_ref.at[pl.ds(SLOTS[s], h), :],
                send_sem=rs_send_sems.at[s],
                recv_sem=rs_recv_sems.at[s],
                device_id=(partners[s],),
                device_id_type=pl.DeviceIdType.MESH,
            )
            rdma.start()
            rdma.wait_recv()
            out_ref[pl.ds(keep_off, h), :] = (
                out_ref[pl.ds(keep_off, h), :]
                + comm_ref[pl.ds(SLOTS[s], h), :]
            )
            rs_rdmas.append(rdma)
            base = keep_off
        for rdma in rs_rdmas:
            rdma.wait_send()

        all_partner_barrier(barrier_sem)

        ag_rdmas = []
        for s in reversed(range(N_STEPS)):
            h = SIZES[s]
            rdma = pltpu.make_async_remote_copy(
                src_ref=out_ref.at[pl.ds(base, h), :],
                dst_ref=out_ref.at[pl.ds(base, h), :],
                send_sem=ag_send_sems.at[s],
                recv_sem=ag_recv_sems.at[s],
                device_id=(partners[s],),
                device_id_type=pl.DeviceIdType.MESH,
            )
            rdma.start()
            rdma.wait_recv()
            ag_rdmas.append(rdma)
            base = base - bits[s] * h
        for rdma in ag_rdmas:
            rdma.wait_send()

    return pl.pallas_call(
        body,
        out_shape=jax.ShapeDtypeStruct((T, H), jnp.float32),
        in_specs=[
            pl.BlockSpec(memory_space=pltpu.VMEM),
            pl.BlockSpec(memory_space=pltpu.VMEM),
            pl.BlockSpec(memory_space=pltpu.VMEM),
            pl.BlockSpec(memory_space=pltpu.VMEM),
        ],
        out_specs=pl.BlockSpec(memory_space=pltpu.VMEM),
        scratch_shapes=[
            pltpu.VMEM((sum(SIZES), H), jnp.float32),
            pltpu.SemaphoreType.DMA((N_STEPS,)),
            pltpu.SemaphoreType.DMA((N_STEPS,)),
            pltpu.SemaphoreType.DMA((N_STEPS,)),
            pltpu.SemaphoreType.DMA((N_STEPS,)),
        ],
        compiler_params=pltpu.CompilerParams(collective_id=0),
    )(x, router_W, route_idx, expert_W)


# device time: 73765 ns/iter; 3.0711x vs baseline; 1.7364x over previous
import os

import jax
import jax.numpy as jnp
from jax import lax
from jax.experimental import pallas as pl
from jax.experimental.pallas import tpu as pltpu

N_DEV = 32
BITS_A = (0, 3, 1, 2, 4)
BITS_B = (3, 0, 2, 1, 4)
N_STEPS = 5
SIZES = tuple(512 >> s for s in range(N_STEPS))
SLOTS = tuple(sum(SIZES[:s]) for s in range(N_STEPS))
STAGE_ROWS = sum(SIZES)


def kernel(x, router_W, route_idx, expert_W):
    T, D = x.shape
    E_local, _, H = expert_W.shape
    HC = H // 2

    def body(x_ref, rw_ref, idx_ref, ew_ref, out_ref,
             sbuf_a, sbuf_b, comm_a, comm_b,
             rsa_s, rsa_r, rsb_s, rsb_r, aga_s, aga_r, agb_s, agb_r):
        my = lax.axis_index("i")
        partners_a = [lax.bitwise_xor(my, 1 << b) for b in BITS_A]
        partners_b = [lax.bitwise_xor(my, 1 << b) for b in BITS_B]
        bits_a = [lax.bitwise_and(lax.shift_right_logical(my, b), 1)
                  for b in BITS_A]
        bits_b = [lax.bitwise_and(lax.shift_right_logical(my, b), 1)
                  for b in BITS_B]

        def all_partner_barrier(sem):
            for p in partners_a:
                pl.semaphore_signal(
                    sem, inc=1, device_id=(p,),
                    device_id_type=pl.DeviceIdType.MESH,
                )
            pl.semaphore_wait(sem, N_STEPS)

        barrier_sem = pltpu.get_barrier_semaphore()
        all_partner_barrier(barrier_sem)

        probe = os.environ.get("PROBE", "")

        xv = x_ref[:, :]
        scores = jnp.dot(xv, rw_ref[:, :],
                         preferred_element_type=jnp.float32)
        m = jnp.max(scores, axis=1, keepdims=True)
        p = jnp.exp(scores - m)
        p = p / jnp.sum(p, axis=1, keepdims=True)

        e0 = idx_ref[:, 0:1]
        e1 = idx_ref[:, 1:2]
        eid = lax.broadcasted_iota(jnp.int32, scores.shape, 1)
        g0 = jnp.sum(jnp.where(eid == e0, p, 0.0), axis=1, keepdims=True)
        g1 = jnp.sum(jnp.where(eid == e1, p, 0.0), axis=1, keepdims=True)
        gs = g0 + g1
        w0 = g0 / gs
        w1 = g1 / gs

        if probe == "comm_only":
            out_ref[:, :] = jnp.broadcast_to(w0, (T, H))
        else:
            acc = jnp.zeros((T, H), jnp.float32)
            for le in range(E_local):
                ge = my * E_local + le
                w = (jnp.where(e0 == ge, w0, 0.0)
                     + jnp.where(e1 == ge, w1, 0.0))
                acc = acc + jnp.dot(w * xv, ew_ref[le],
                                    preferred_element_type=jnp.float32)
            out_ref[:, :] = acc

        if probe == "compute_only":
            return

        base_a = jnp.int32(0)
        base_b = jnp.int32(0)
        in_flight = []
        for s in range(N_STEPS):
            h = SIZES[s]
            sl = SLOTS[s]
            ka = base_a + bits_a[s] * h
            sa = base_a + (1 - bits_a[s]) * h
            kb = base_b + bits_b[s] * h
            sb = base_b + (1 - bits_b[s]) * h

            sbuf_a[pl.ds(sl, h), :] = out_ref[pl.ds(sa, h), 0:HC].astype(
                jnp.bfloat16)
            ra = pltpu.make_async_remote_copy(
                src_ref=sbuf_a.at[pl.ds(sl, h), :],
                dst_ref=comm_a.at[pl.ds(sl, h), :],
                send_sem=rsa_s.at[s], recv_sem=rsa_r.at[s],
                device_id=(partners_a[s],),
                device_id_type=pl.DeviceIdType.MESH,
            )
            ra.start()

            sbuf_b[pl.ds(sl, h), :] = out_ref[pl.ds(sb, h), HC:H].astype(
                jnp.bfloat16)
            rb = pltpu.make_async_remote_copy(
                src_ref=sbuf_b.at[pl.ds(sl, h), :],
                dst_ref=comm_b.at[pl.ds(sl, h), :],
                send_sem=rsb_s.at[s], recv_sem=rsb_r.at[s],
                device_id=(partners_b[s],),
                device_id_type=pl.DeviceIdType.MESH,
            )
            rb.start()

            ra.wait_recv()
            out_ref[pl.ds(ka, h), 0:HC] = (
                out_ref[pl.ds(ka, h), 0:HC]
                + comm_a[pl.ds(sl, h), :].astype(jnp.float32)
            )
            rb.wait_recv()
            out_ref[pl.ds(kb, h), HC:H] = (
                out_ref[pl.ds(kb, h), HC:H]
                + comm_b[pl.ds(sl, h), :].astype(jnp.float32)
            )
            base_a = ka
            base_b = kb
            in_flight += [ra, rb]
        for r in in_flight:
            r.wait_send()

        all_partner_barrier(barrier_sem)

        in_flight = []
        for s in reversed(range(N_STEPS)):
            h = SIZES[s]
            sl = SLOTS[s]

            sbuf_a[pl.ds(sl, h), :] = out_ref[pl.ds(base_a, h), 0:HC].astype(
                jnp.bfloat16)
            ra = pltpu.make_async_remote_copy(
                src_ref=sbuf_a.at[pl.ds(sl, h), :],
                dst_ref=comm_a.at[pl.ds(sl, h), :],
                send_sem=aga_s.at[s], recv_sem=aga_r.at[s],
                device_id=(partners_a[s],),
                device_id_type=pl.DeviceIdType.MESH,
            )
            ra.start()

            sbuf_b[pl.ds(sl, h), :] = out_ref[pl.ds(base_b, h), HC:H].astype(
                jnp.bfloat16)
            rb = pltpu.make_async_remote_copy(
                src_ref=sbuf_b.at[pl.ds(sl, h), :],
                dst_ref=comm_b.at[pl.ds(sl, h), :],
                send_sem=agb_s.at[s], recv_sem=agb_r.at[s],
                device_id=(partners_b[s],),
                device_id_type=pl.DeviceIdType.MESH,
            )
            rb.start()

            recv_a = base_a + (1 - 2 * bits_a[s]) * h
            recv_b = base_b + (1 - 2 * bits_b[s]) * h
            ra.wait_recv()
            out_ref[pl.ds(recv_a, h), 0:HC] = comm_a[
                pl.ds(sl, h), :].astype(jnp.float32)
            rb.wait_recv()
            out_ref[pl.ds(recv_b, h), HC:H] = comm_b[
                pl.ds(sl, h), :].astype(jnp.float32)
            base_a = base_a - bits_a[s] * h
            base_b = base_b - bits_b[s] * h
            in_flight += [ra, rb]
        for r in in_flight:
            r.wait_send()

    return pl.pallas_call(
        body,
        out_shape=jax.ShapeDtypeStruct((T, H), jnp.float32),
        in_specs=[pl.BlockSpec(memory_space=pltpu.VMEM)] * 4,
        out_specs=pl.BlockSpec(memory_space=pltpu.VMEM),
        scratch_shapes=[
            pltpu.VMEM((STAGE_ROWS, H // 2), jnp.bfloat16),
            pltpu.VMEM((STAGE_ROWS, H // 2), jnp.bfloat16),
            pltpu.VMEM((STAGE_ROWS, H // 2), jnp.bfloat16),
            pltpu.VMEM((STAGE_ROWS, H // 2), jnp.bfloat16),
            pltpu.SemaphoreType.DMA((N_STEPS,)),
            pltpu.SemaphoreType.DMA((N_STEPS,)),
            pltpu.SemaphoreType.DMA((N_STEPS,)),
            pltpu.SemaphoreType.DMA((N_STEPS,)),
            pltpu.SemaphoreType.DMA((N_STEPS,)),
            pltpu.SemaphoreType.DMA((N_STEPS,)),
            pltpu.SemaphoreType.DMA((N_STEPS,)),
            pltpu.SemaphoreType.DMA((N_STEPS,)),
        ],
        compiler_params=pltpu.CompilerParams(collective_id=0),
    )(x, router_W, route_idx, expert_W)


# device time: 73708 ns/iter; 3.0735x vs baseline; 1.0008x over previous
import os

import jax
import jax.numpy as jnp
from jax import lax
from jax.experimental import pallas as pl
from jax.experimental.pallas import tpu as pltpu

N_DEV = 32
BITS_A = (0, 3, 1, 2, 4)
BITS_B = (3, 0, 2, 1, 4)
N_STEPS = 5
SIZES = tuple(512 >> s for s in range(N_STEPS))
SLOTS = tuple(sum(SIZES[:s]) for s in range(N_STEPS))
STAGE_ROWS = sum(SIZES)


def kernel(x, router_W, route_idx, expert_W):
    T, D = x.shape
    E_local, _, H = expert_W.shape
    HC = H // 2

    def body(x_ref, rw_ref, idx_ref, ew_ref, out_ref,
             sbuf_a, sbuf_b, comm_a, comm_b,
             agsbuf_a, agsbuf_b, agcomm_a, agcomm_b,
             rsa_s, rsa_r, rsb_s, rsb_r, aga_s, aga_r, agb_s, agb_r):
        my = lax.axis_index("i")
        partners_a = [lax.bitwise_xor(my, 1 << b) for b in BITS_A]
        partners_b = [lax.bitwise_xor(my, 1 << b) for b in BITS_B]
        bits_a = [lax.bitwise_and(lax.shift_right_logical(my, b), 1)
                  for b in BITS_A]
        bits_b = [lax.bitwise_and(lax.shift_right_logical(my, b), 1)
                  for b in BITS_B]

        def all_partner_barrier(sem):
            for p in partners_a:
                pl.semaphore_signal(
                    sem, inc=1, device_id=(p,),
                    device_id_type=pl.DeviceIdType.MESH,
                )
            pl.semaphore_wait(sem, N_STEPS)

        barrier_sem = pltpu.get_barrier_semaphore()
        all_partner_barrier(barrier_sem)

        probe = os.environ.get("PROBE", "")

        xv = x_ref[:, :]
        scores = jnp.dot(xv, rw_ref[:, :],
                         preferred_element_type=jnp.float32)
        m = jnp.max(scores, axis=1, keepdims=True)
        p = jnp.exp(scores - m)
        p = p / jnp.sum(p, axis=1, keepdims=True)

        e0 = idx_ref[:, 0:1]
        e1 = idx_ref[:, 1:2]
        eid = lax.broadcasted_iota(jnp.int32, scores.shape, 1)
        g0 = jnp.sum(jnp.where(eid == e0, p, 0.0), axis=1, keepdims=True)
        g1 = jnp.sum(jnp.where(eid == e1, p, 0.0), axis=1, keepdims=True)
        gs = g0 + g1
        w0 = g0 / gs
        w1 = g1 / gs

        if probe == "comm_only":
            out_ref[:, :] = jnp.broadcast_to(w0, (T, H))
        else:
            acc = jnp.zeros((T, H), jnp.float32)
            for le in range(E_local):
                ge = my * E_local + le
                w = (jnp.where(e0 == ge, w0, 0.0)
                     + jnp.where(e1 == ge, w1, 0.0))
                acc = acc + jnp.dot(w * xv, ew_ref[le],
                                    preferred_element_type=jnp.float32)
            out_ref[:, :] = acc

        if probe == "compute_only":
            return

        base_a = jnp.int32(0)
        base_b = jnp.int32(0)
        in_flight = []
        for s in range(N_STEPS):
            h = SIZES[s]
            sl = SLOTS[s]
            ka = base_a + bits_a[s] * h
            sa = base_a + (1 - bits_a[s]) * h
            kb = base_b + bits_b[s] * h
            sb = base_b + (1 - bits_b[s]) * h

            sbuf_a[pl.ds(sl, h), :] = out_ref[pl.ds(sa, h), 0:HC].astype(
                jnp.bfloat16)
            ra = pltpu.make_async_remote_copy(
                src_ref=sbuf_a.at[pl.ds(sl, h), :],
                dst_ref=comm_a.at[pl.ds(sl, h), :],
                send_sem=rsa_s.at[s], recv_sem=rsa_r.at[s],
                device_id=(partners_a[s],),
                device_id_type=pl.DeviceIdType.MESH,
            )
            ra.start()

            sbuf_b[pl.ds(sl, h), :] = out_ref[pl.ds(sb, h), HC:H].astype(
                jnp.bfloat16)
            rb = pltpu.make_async_remote_copy(
                src_ref=sbuf_b.at[pl.ds(sl, h), :],
                dst_ref=comm_b.at[pl.ds(sl, h), :],
                send_sem=rsb_s.at[s], recv_sem=rsb_r.at[s],
                device_id=(partners_b[s],),
                device_id_type=pl.DeviceIdType.MESH,
            )
            rb.start()

            ra.wait_recv()
            out_ref[pl.ds(ka, h), 0:HC] = (
                out_ref[pl.ds(ka, h), 0:HC]
                + comm_a[pl.ds(sl, h), :].astype(jnp.float32)
            )
            rb.wait_recv()
            out_ref[pl.ds(kb, h), HC:H] = (
                out_ref[pl.ds(kb, h), HC:H]
                + comm_b[pl.ds(sl, h), :].astype(jnp.float32)
            )
            base_a = ka
            base_b = kb
            in_flight += [ra, rb]

        all_partner_barrier(barrier_sem)

        for s in reversed(range(N_STEPS)):
            h = SIZES[s]
            sl = SLOTS[s]

            agsbuf_a[pl.ds(sl, h), :] = out_ref[
                pl.ds(base_a, h), 0:HC].astype(jnp.bfloat16)
            ra = pltpu.make_async_remote_copy(
                src_ref=agsbuf_a.at[pl.ds(sl, h), :],
                dst_ref=agcomm_a.at[pl.ds(sl, h), :],
                send_sem=aga_s.at[s], recv_sem=aga_r.at[s],
                device_id=(partners_a[s],),
                device_id_type=pl.DeviceIdType.MESH,
            )
            ra.start()

            agsbuf_b[pl.ds(sl, h), :] = out_ref[
                pl.ds(base_b, h), HC:H].astype(jnp.bfloat16)
            rb = pltpu.make_async_remote_copy(
                src_ref=agsbuf_b.at[pl.ds(sl, h), :],
                dst_ref=agcomm_b.at[pl.ds(sl, h), :],
                send_sem=agb_s.at[s], recv_sem=agb_r.at[s],
                device_id=(partners_b[s],),
                device_id_type=pl.DeviceIdType.MESH,
            )
            rb.start()

            recv_a = base_a + (1 - 2 * bits_a[s]) * h
            recv_b = base_b + (1 - 2 * bits_b[s]) * h
            ra.wait_recv()
            out_ref[pl.ds(recv_a, h), 0:HC] = agcomm_a[
                pl.ds(sl, h), :].astype(jnp.float32)
            rb.wait_recv()
            out_ref[pl.ds(recv_b, h), HC:H] = agcomm_b[
                pl.ds(sl, h), :].astype(jnp.float32)
            base_a = base_a - bits_a[s] * h
            base_b = base_b - bits_b[s] * h
            in_flight += [ra, rb]
        for r in in_flight:
            r.wait_send()

    return pl.pallas_call(
        body,
        out_shape=jax.ShapeDtypeStruct((T, H), jnp.float32),
        in_specs=[pl.BlockSpec(memory_space=pltpu.VMEM)] * 4,
        out_specs=pl.BlockSpec(memory_space=pltpu.VMEM),
        scratch_shapes=[
            pltpu.VMEM((STAGE_ROWS, H // 2), jnp.bfloat16),
            pltpu.VMEM((STAGE_ROWS, H // 2), jnp.bfloat16),
            pltpu.VMEM((STAGE_ROWS, H // 2), jnp.bfloat16),
            pltpu.VMEM((STAGE_ROWS, H // 2), jnp.bfloat16),
            pltpu.VMEM((STAGE_ROWS, H // 2), jnp.bfloat16),
            pltpu.VMEM((STAGE_ROWS, H // 2), jnp.bfloat16),
            pltpu.VMEM((STAGE_ROWS, H // 2), jnp.bfloat16),
            pltpu.VMEM((STAGE_ROWS, H // 2), jnp.bfloat16),
            pltpu.SemaphoreType.DMA((N_STEPS,)),
            pltpu.SemaphoreType.DMA((N_STEPS,)),
            pltpu.SemaphoreType.DMA((N_STEPS,)),
            pltpu.SemaphoreType.DMA((N_STEPS,)),
            pltpu.SemaphoreType.DMA((N_STEPS,)),
            pltpu.SemaphoreType.DMA((N_STEPS,)),
            pltpu.SemaphoreType.DMA((N_STEPS,)),
            pltpu.SemaphoreType.DMA((N_STEPS,)),
        ],
        compiler_params=pltpu.CompilerParams(collective_id=0),
    )(x, router_W, route_idx, expert_W)


# device time: 73109 ns/iter; 3.0987x vs baseline; 1.0082x over previous
import os

import jax
import jax.numpy as jnp
from jax import lax
from jax.experimental import pallas as pl
from jax.experimental.pallas import tpu as pltpu

N_DEV = 32
BITS_A = (0, 3, 1, 2, 4)
BITS_B = (3, 0, 2, 1, 4)
N_STEPS = 5
SIZES = tuple(512 >> s for s in range(N_STEPS))
SLOTS = tuple(sum(SIZES[:s]) for s in range(N_STEPS))
STAGE_ROWS = sum(SIZES)


def kernel(x, router_W, route_idx, expert_W):
    T, D = x.shape
    E_local, _, H = expert_W.shape
    HC = H // 2

    def body(x_ref, rw_ref, idx_ref, ew_ref, out_ref,
             sbuf_a, sbuf_b, comm_a, comm_b,
             agsbuf_a, agsbuf_b, agcomm_a, agcomm_b,
             rsa_s, rsa_r, rsb_s, rsb_r, aga_s, aga_r, agb_s, agb_r):
        my = lax.axis_index("i")
        partners_a = [lax.bitwise_xor(my, 1 << b) for b in BITS_A]
        partners_b = [lax.bitwise_xor(my, 1 << b) for b in BITS_B]
        bits_a = [lax.bitwise_and(lax.shift_right_logical(my, b), 1)
                  for b in BITS_A]
        bits_b = [lax.bitwise_and(lax.shift_right_logical(my, b), 1)
                  for b in BITS_B]

        def signal_partners(sem):
            for p in partners_a:
                pl.semaphore_signal(
                    sem, inc=1, device_id=(p,),
                    device_id_type=pl.DeviceIdType.MESH,
                )

        barrier_sem = pltpu.get_barrier_semaphore()
        signal_partners(barrier_sem)
        pl.semaphore_wait(barrier_sem, N_STEPS)

        probe = os.environ.get("PROBE", "")

        xv = x_ref[:, :]
        scores = jnp.dot(xv, rw_ref[:, :],
                         preferred_element_type=jnp.float32)
        m = jnp.max(scores, axis=1, keepdims=True)
        p = jnp.exp(scores - m)
        p = p / jnp.sum(p, axis=1, keepdims=True)

        e0 = idx_ref[:, 0:1]
        e1 = idx_ref[:, 1:2]
        eid = lax.broadcasted_iota(jnp.int32, scores.shape, 1)
        g0 = jnp.sum(jnp.where(eid == e0, p, 0.0), axis=1, keepdims=True)
        g1 = jnp.sum(jnp.where(eid == e1, p, 0.0), axis=1, keepdims=True)
        gs = g0 + g1
        w0 = g0 / gs
        w1 = g1 / gs

        ws = []
        for le in range(E_local):
            ge = my * E_local + le
            ws.append(jnp.where(e0 == ge, w0, 0.0)
                      + jnp.where(e1 == ge, w1, 0.0))
        wx = [w * xv for w in ws]

        def compute_half(lo, hi):
            acc = jnp.zeros((T, hi - lo), jnp.float32)
            for le in range(E_local):
                acc = acc + jnp.dot(wx[le], ew_ref[le, :, lo:hi],
                                    preferred_element_type=jnp.float32)
            return acc

        comm_only = probe == "comm_only"
        compute_only = probe == "compute_only"

        def rs_descriptor(s, send_off, half):
            sbuf, comm, ssem, rsem, partners = (
                (sbuf_a, comm_a, rsa_s, rsa_r, partners_a) if half == 0
                else (sbuf_b, comm_b, rsb_s, rsb_r, partners_b))
            h = SIZES[s]
            sl = SLOTS[s]
            lo = half * HC
            sbuf[pl.ds(sl, h), :] = out_ref[
                pl.ds(send_off, h), lo:lo + HC].astype(jnp.bfloat16)
            return pltpu.make_async_remote_copy(
                src_ref=sbuf.at[pl.ds(sl, h), :],
                dst_ref=comm.at[pl.ds(sl, h), :],
                send_sem=ssem.at[s], recv_sem=rsem.at[s],
                device_id=(partners[s],),
                device_id_type=pl.DeviceIdType.MESH,
            )

        in_flight = []

        if comm_only:
            out_ref[:, 0:HC] = jnp.broadcast_to(w0, (T, HC))
        else:
            out_ref[:, 0:HC] = compute_half(0, HC)
        base_a = jnp.int32(0)
        if not compute_only:
            ra = rs_descriptor(0, base_a + (1 - bits_a[0]) * SIZES[0], 0)
            ra.start()
            in_flight.append(ra)

        if comm_only:
            out_ref[:, HC:H] = jnp.broadcast_to(w0, (T, HC))
        else:
            out_ref[:, HC:H] = compute_half(HC, H)

        if compute_only:
            return

        base_b = jnp.int32(0)
        rb = rs_descriptor(0, base_b + (1 - bits_b[0]) * SIZES[0], 1)
        rb.start()
        in_flight.append(rb)

        for s in range(N_STEPS):
            h = SIZES[s]
            sl = SLOTS[s]
            ka = base_a + bits_a[s] * h
            kb = base_b + bits_b[s] * h

            ra.wait_recv()
            out_ref[pl.ds(ka, h), 0:HC] = (
                out_ref[pl.ds(ka, h), 0:HC]
                + comm_a[pl.ds(sl, h), :].astype(jnp.float32)
            )
            base_a = ka
            if s + 1 < N_STEPS:
                ra = rs_descriptor(
                    s + 1, base_a + (1 - bits_a[s + 1]) * SIZES[s + 1], 0)
                ra.start()
                in_flight.append(ra)

            rb.wait_recv()
            out_ref[pl.ds(kb, h), HC:H] = (
                out_ref[pl.ds(kb, h), HC:H]
                + comm_b[pl.ds(sl, h), :].astype(jnp.float32)
            )
            base_b = kb
            if s + 1 < N_STEPS:
                rb = rs_descriptor(
                    s + 1, base_b + (1 - bits_b[s + 1]) * SIZES[s + 1], 1)
                rb.start()
                in_flight.append(rb)

        signal_partners(barrier_sem)

        def ag_descriptor(s, base, half):
            sbuf, comm, ssem, rsem, partners = (
                (agsbuf_a, agcomm_a, aga_s, aga_r, partners_a) if half == 0
                else (agsbuf_b, agcomm_b, agb_s, agb_r, partners_b))
            h = SIZES[s]
            sl = SLOTS[s]
            lo = half * HC
            sbuf[pl.ds(sl, h), :] = out_ref[
                pl.ds(base, h), lo:lo + HC].astype(jnp.bfloat16)
            return pltpu.make_async_remote_copy(
                src_ref=sbuf.at[pl.ds(sl, h), :],
                dst_ref=comm.at[pl.ds(sl, h), :],
                send_sem=ssem.at[s], recv_sem=rsem.at[s],
                device_id=(partners[s],),
                device_id_type=pl.DeviceIdType.MESH,
            )

        pl.semaphore_wait(barrier_sem, N_STEPS)

        for s in reversed(range(N_STEPS)):
            h = SIZES[s]
            sl = SLOTS[s]
            ra = ag_descriptor(s, base_a, 0)
            ra.start()
            rb = ag_descriptor(s, base_b, 1)
            rb.start()
            recv_a = base_a + (1 - 2 * bits_a[s]) * h
            recv_b = base_b + (1 - 2 * bits_b[s]) * h
            ra.wait_recv()
            out_ref[pl.ds(recv_a, h), 0:HC] = agcomm_a[
                pl.ds(sl, h), :].astype(jnp.float32)
            rb.wait_recv()
            out_ref[pl.ds(recv_b, h), HC:H] = agcomm_b[
                pl.ds(sl, h), :].astype(jnp.float32)
            base_a = base_a - bits_a[s] * h
            base_b = base_b - bits_b[s] * h
            in_flight += [ra, rb]

        for r in in_flight:
            r.wait_send()

    return pl.pallas_call(
        body,
        out_shape=jax.ShapeDtypeStruct((T, H), jnp.float32),
        in_specs=[pl.BlockSpec(memory_space=pltpu.VMEM)] * 4,
        out_specs=pl.BlockSpec(memory_space=pltpu.VMEM),
        scratch_shapes=[
            pltpu.VMEM((STAGE_ROWS, H // 2), jnp.bfloat16),
            pltpu.VMEM((STAGE_ROWS, H // 2), jnp.bfloat16),
            pltpu.VMEM((STAGE_ROWS, H // 2), jnp.bfloat16),
            pltpu.VMEM((STAGE_ROWS, H // 2), jnp.bfloat16),
            pltpu.VMEM((STAGE_ROWS, H // 2), jnp.bfloat16),
            pltpu.VMEM((STAGE_ROWS, H // 2), jnp.bfloat16),
            pltpu.VMEM((STAGE_ROWS, H // 2), jnp.bfloat16),
            pltpu.VMEM((STAGE_ROWS, H // 2), jnp.bfloat16),
            pltpu.SemaphoreType.DMA((N_STEPS,)),
            pltpu.SemaphoreType.DMA((N_STEPS,)),
            pltpu.SemaphoreType.DMA((N_STEPS,)),
            pltpu.SemaphoreType.DMA((N_STEPS,)),
            pltpu.SemaphoreType.DMA((N_STEPS,)),
            pltpu.SemaphoreType.DMA((N_STEPS,)),
            pltpu.SemaphoreType.DMA((N_STEPS,)),
            pltpu.SemaphoreType.DMA((N_STEPS,)),
        ],
        compiler_params=pltpu.CompilerParams(collective_id=0),
    )(x, router_W, route_idx, expert_W)
